# baseline (speedup 1.0000x reference)
import functools

import jax
import jax.numpy as jnp
from jax.experimental import pallas as pl
from jax.experimental.pallas import tpu as pltpu

B_TOPK = 200
BATCH = 8
CH = 96
HW = 224 * 224
ROWS = 768
CHUNK = 24
NCHUNKS = ROWS // CHUNK
CPB = CH // CHUNK
NBUF = 4


def _body(pred_hbm, target_hbm, out_ref, pbuf, tbuf, acc, sems):
    def copy_in(i, slot):
        row = i * CHUNK
        p = pltpu.make_async_copy(
            pred_hbm.at[pl.ds(row, CHUNK), :], pbuf.at[slot], sems.at[0, slot]
        )
        t = pltpu.make_async_copy(
            target_hbm.at[pl.ds(row, CHUNK), :], tbuf.at[slot], sems.at[1, slot]
        )
        return p, t

    for i in range(NBUF):
        p, t = copy_in(i, i)
        p.start()
        t.start()

    def step(i, _):
        slot = jax.lax.rem(i, NBUF)
        p, t = copy_in(i, slot)
        p.wait()
        t.wait()
        d = tbuf[slot] - pbuf[slot]
        s = jnp.sum(d * d, axis=0, keepdims=True)
        b = jax.lax.div(i, CPB)
        c = jax.lax.rem(i, CPB)

        @pl.when(c == 0)
        def _init():
            acc[pl.ds(b, 1), :] = s

        @pl.when(c != 0)
        def _accum():
            acc[pl.ds(b, 1), :] += s

        @pl.when(i + NBUF < NCHUNKS)
        def _next():
            pn, tn = copy_in(i + NBUF, slot)
            pn.start()
            tn.start()

        return 0

    jax.lax.fori_loop(0, NCHUNKS, step, 0)

    diff = acc[...]
    bits = jax.lax.bitcast_convert_type(diff, jnp.int32)

    lo0 = jnp.zeros((BATCH, 1), jnp.int32)
    hi0 = jnp.max(bits, axis=1, keepdims=True) + 1

    def bis(_, carry):
        lo, hi = carry
        mid = lo + ((hi - lo) >> 1)
        cnt = jnp.sum((bits >= mid).astype(jnp.int32), axis=1, keepdims=True)
        take = cnt >= B_TOPK
        return jnp.where(take, mid, lo), jnp.where(take, hi, mid)

    lo, _ = jax.lax.fori_loop(0, 32, bis, (lo0, hi0))

    vk = jax.lax.bitcast_convert_type(lo, jnp.float32)
    gt = diff > vk
    cnt_gt = jnp.sum(gt.astype(jnp.float32), axis=1, keepdims=True)
    sum_gt = jnp.sum(jnp.where(gt, diff, 0.0), axis=1, keepdims=True)
    row_top = sum_gt + (B_TOPK - cnt_gt) * vk
    out_ref[...] = jnp.sum(row_top).reshape(1, 1) / (BATCH * B_TOPK)


@jax.jit
def kernel(pred, target):
    pred = pred.reshape(ROWS, HW)
    target = target.reshape(ROWS, HW)

    loss = pl.pallas_call(
        _body,
        in_specs=[
            pl.BlockSpec(memory_space=pl.ANY),
            pl.BlockSpec(memory_space=pl.ANY),
        ],
        out_shape=jax.ShapeDtypeStruct((1, 1), jnp.float32),
        scratch_shapes=[
            pltpu.VMEM((NBUF, CHUNK, HW), jnp.float32),
            pltpu.VMEM((NBUF, CHUNK, HW), jnp.float32),
            pltpu.VMEM((BATCH, HW), jnp.float32),
            pltpu.SemaphoreType.DMA((2, NBUF)),
        ],
    )(pred, target)
    return loss.reshape(())


# speedup vs baseline: 3.8281x; 3.8281x over previous
import functools

import jax
import jax.numpy as jnp
from jax.experimental import pallas as pl
from jax.experimental.pallas import tpu as pltpu

B_TOPK = 200
BATCH = 8
CH = 96
H = 224
W = 224
N_H_TILES = 4
H_TILE = H // N_H_TILES


def _diff_body(pred_ref, target_ref, out_ref):
    d = target_ref[0] - pred_ref[0]
    out_ref[0] = jnp.sum(d * d, axis=0)


def _topk_mean_body(diff_ref, out_ref):
    diff = diff_ref[...]
    bits = jax.lax.bitcast_convert_type(diff, jnp.int32)

    lo0 = jnp.zeros((BATCH, 1, 1), jnp.int32)
    hi0 = jnp.max(bits, axis=(1, 2), keepdims=True) + 1

    def bis(_, carry):
        lo, hi = carry
        mid = lo + ((hi - lo) >> 1)
        cnt = jnp.sum((bits >= mid).astype(jnp.int32), axis=(1, 2),
                      keepdims=True)
        take = cnt >= B_TOPK
        return jnp.where(take, mid, lo), jnp.where(take, hi, mid)

    lo, _ = jax.lax.fori_loop(0, 32, bis, (lo0, hi0))

    vk = jax.lax.bitcast_convert_type(lo, jnp.float32)
    gt = diff > vk
    cnt_gt = jnp.sum(gt.astype(jnp.float32), axis=(1, 2), keepdims=True)
    sum_gt = jnp.sum(jnp.where(gt, diff, 0.0), axis=(1, 2), keepdims=True)
    row_top = sum_gt + (B_TOPK - cnt_gt) * vk
    out_ref[...] = jnp.sum(row_top).reshape(1, 1) / (BATCH * B_TOPK)


@jax.jit
def kernel(pred, target):
    spec = pl.BlockSpec((1, CH, H_TILE, W), lambda b, t: (b, 0, t, 0))
    diff = pl.pallas_call(
        _diff_body,
        grid=(BATCH, N_H_TILES),
        in_specs=[spec, spec],
        out_specs=pl.BlockSpec((1, H_TILE, W), lambda b, t: (b, t, 0)),
        out_shape=jax.ShapeDtypeStruct((BATCH, H, W), jnp.float32),
        compiler_params=pltpu.CompilerParams(
            dimension_semantics=("parallel", "parallel"),
        ),
    )(pred, target)

    loss = pl.pallas_call(
        _topk_mean_body,
        out_shape=jax.ShapeDtypeStruct((1, 1), jnp.float32),
    )(diff)
    return loss.reshape(())


# speedup vs baseline: 3.9090x; 1.0211x over previous
import functools

import jax
import jax.numpy as jnp
from jax.experimental import pallas as pl
from jax.experimental.pallas import tpu as pltpu

B_TOPK = 200
BATCH = 8
CH = 96
H = 224
W = 224
N_H_TILES = 4
H_TILE = H // N_H_TILES


def _body(pred_ref, target_ref, out_ref, acc):
    b = pl.program_id(0)
    t = pl.program_id(1)
    d = target_ref[0] - pred_ref[0]
    acc[b, pl.ds(t * H_TILE, H_TILE), :] = jnp.sum(d * d, axis=0)

    @pl.when((b == BATCH - 1) & (t == N_H_TILES - 1))
    def _select():
        diff = acc[...]
        bits = jax.lax.bitcast_convert_type(diff, jnp.int32)

        lo0 = jnp.zeros((BATCH, 1, 1), jnp.int32)
        hi0 = jnp.max(bits, axis=(1, 2), keepdims=True) + 1

        def bis(_, carry):
            lo, hi = carry
            mid = lo + ((hi - lo) >> 1)
            cnt = jnp.sum((bits >= mid).astype(jnp.int32), axis=(1, 2),
                          keepdims=True)
            take = cnt >= B_TOPK
            return jnp.where(take, mid, lo), jnp.where(take, hi, mid)

        lo, _ = jax.lax.fori_loop(0, 32, bis, (lo0, hi0))

        vk = jax.lax.bitcast_convert_type(lo, jnp.float32)
        gt = diff > vk
        cnt_gt = jnp.sum(gt.astype(jnp.float32), axis=(1, 2), keepdims=True)
        sum_gt = jnp.sum(jnp.where(gt, diff, 0.0), axis=(1, 2), keepdims=True)
        row_top = sum_gt + (B_TOPK - cnt_gt) * vk
        out_ref[...] = jnp.sum(row_top).reshape(1, 1) / (BATCH * B_TOPK)


@jax.jit
def kernel(pred, target):
    spec = pl.BlockSpec((1, CH, H_TILE, W), lambda b, t: (b, 0, t, 0))
    loss = pl.pallas_call(
        _body,
        grid=(BATCH, N_H_TILES),
        in_specs=[spec, spec],
        out_specs=pl.BlockSpec((1, 1), lambda b, t: (0, 0)),
        out_shape=jax.ShapeDtypeStruct((1, 1), jnp.float32),
        scratch_shapes=[pltpu.VMEM((BATCH, H, W), jnp.float32)],
    )(pred, target)
    return loss.reshape(())
